# baseline (speedup 1.0000x reference)
import jax
import jax.numpy as jnp
from jax.experimental import pallas as pl
from jax.experimental.pallas import tpu as pltpu

_G = 8
_R1 = 1024
_WP2 = 24
_R2 = 432


def _round_up(x, m):
    return (x + m - 1) // m * m


def _fused_conv_kernel(x5_ref, w1_ref, b1_ref, w2_ref, b2_ref, o_ref,
                       patch1, acc1, x2i, acc2):
    x2i[...] = jnp.zeros(x2i.shape, x2i.dtype)

    for g in range(_G):
        for kh in range(5):
            patch1[:, pl.ds(kh * 64, 64)] = \
                x5_ref[g, pl.ds(kh * 32, 896), :]
        a1 = jnp.dot(patch1[...], w1_ref[...],
                     preferred_element_type=jnp.float32)
        acc1[...] = jnp.maximum(a1 + b1_ref[...], 0.0)

        for i in range(14):
            r0 = (2 * i) * 32
            r1 = r0 + 32
            p = jnp.maximum(
                jnp.maximum(acc1[pl.ds(r0, 14, 2), :],
                            acc1[pl.ds(r0 + 1, 14, 2), :]),
                jnp.maximum(acc1[pl.ds(r1, 14, 2), :],
                            acc1[pl.ds(r1 + 1, 14, 2), :]))
            pb = p.astype(jnp.bfloat16)
            for s in range(2):
                half = pb[:, s * 64:(s + 1) * 64]
                for kw in range(5):
                    x2i[s, pl.ds((i + 2) * _WP2 + 2 - kw, 14),
                        pl.ds(kw * 64, 64)] = half

        for s in range(2):
            acc = None
            for kh in range(5):
                d = jnp.dot(x2i[s, pl.ds(kh * _WP2, 336), :],
                            w2_ref[pl.ds(kh * 512, 512), :],
                            preferred_element_type=jnp.float32)
                acc = d if acc is None else acc + d
            acc2[...] = jnp.maximum(acc + b2_ref[...], 0.0)
            for i in range(7):
                r0 = (2 * i) * _WP2
                r1 = r0 + _WP2
                p = jnp.maximum(
                    jnp.maximum(acc2[pl.ds(r0, 7, 2), :],
                                acc2[pl.ds(r0 + 1, 7, 2), :]),
                    jnp.maximum(acc2[pl.ds(r1, 7, 2), :],
                                acc2[pl.ds(r1 + 1, 7, 2), :]))
                o_ref[g, s, pl.ds(i * 7, 7), :] = p.astype(o_ref.dtype)


def _fused_call(x5, w1, b1, w2, b2):
    nblk = x5.shape[0]
    return pl.pallas_call(
        _fused_conv_kernel,
        out_shape=jax.ShapeDtypeStruct((nblk, 2, 49, 128), jnp.bfloat16),
        grid=(nblk // _G,),
        in_specs=[
            pl.BlockSpec((_G, _R1, 64), lambda i: (i, 0, 0)),
            pl.BlockSpec((320, 128), lambda i: (0, 0)),
            pl.BlockSpec((1, 128), lambda i: (0, 0)),
            pl.BlockSpec((2560, 128), lambda i: (0, 0)),
            pl.BlockSpec((1, 128), lambda i: (0, 0)),
        ],
        out_specs=pl.BlockSpec((_G, 2, 49, 128), lambda i: (i, 0, 0, 0)),
        scratch_shapes=[
            pltpu.VMEM((896, 320), jnp.bfloat16),
            pltpu.VMEM((896, 128), jnp.float32),
            pltpu.VMEM((2, _R2, 512), jnp.bfloat16),
            pltpu.VMEM((336, 128), jnp.float32),
        ],
        compiler_params=pltpu.CompilerParams(
            dimension_semantics=("parallel",)),
    )(x5, w1, b1, w2, b2)


def _prep_weights(w1bd, w2bd):
    f32 = jnp.float32
    w1t = w1bd[:200].reshape(25, 8, 128)[:, 0, 0:16].reshape(5, 5, 16)
    t1 = jnp.einsum('hwc,dk->hwdkc', w1t, jnp.eye(8, dtype=f32))
    t1 = t1.reshape(5, 5, 8, 128).reshape(5, 40, 128)
    t1 = jnp.pad(t1, ((0, 0), (0, 24), (0, 0)))
    w1 = t1.reshape(320, 128).astype(jnp.bfloat16)
    w2t = w2bd.reshape(25, 4, 16, 128)[:, 0, :, 0:32].reshape(5, 5, 16, 32)
    t2 = jnp.einsum('hwxy,de->hwdxey', w2t, jnp.eye(4, dtype=f32))
    t2 = t2.reshape(5, 5, 64, 128)
    t2 = jnp.pad(t2, ((0, 0), (0, 3), (0, 0), (0, 0)))
    w2 = t2.reshape(5, 512, 128).reshape(2560, 128).astype(jnp.bfloat16)
    return w1, w2


def _fc_kernel(x_ref, w_ref, b_ref, o_ref):
    o_ref[...] = (jnp.dot(x_ref[...], w_ref[...],
                          preferred_element_type=jnp.float32)
                  + b_ref[...])


def _fc_call(x, w, bias, *, tile_m=512):
    M, K = x.shape
    _, N = w.shape
    tm = min(_round_up(M, 8), tile_m)
    Mp = _round_up(M, tm)
    if Mp != M:
        x = jnp.pad(x, ((0, Mp - M), (0, 0)))
    out = pl.pallas_call(
        _fc_kernel,
        out_shape=jax.ShapeDtypeStruct((Mp, N), jnp.float32),
        grid=(Mp // tm,),
        in_specs=[
            pl.BlockSpec((tm, K), lambda i: (i, 0)),
            pl.BlockSpec((K, N), lambda i: (0, 0)),
            pl.BlockSpec((1, N), lambda i: (0, 0)),
        ],
        out_specs=pl.BlockSpec((tm, N), lambda i: (i, 0)),
        compiler_params=pltpu.CompilerParams(
            dimension_semantics=("parallel",)),
    )(x, w, bias.reshape(1, N))
    return out[:M]


def kernel(w1bd, b1bd, w2bd, b2bd, fc_w, fc_b, x):
    B = x.shape[0]
    bpad = _round_up(B, 8 * _G)
    xb = x[:, 0, :, :]
    if bpad != B:
        xb = jnp.pad(xb, ((0, bpad - B), (0, 0), (0, 0)))
    xb = xb.astype(jnp.bfloat16)
    xp = jnp.pad(xb, ((0, 0), (2, 2), (2, 2)))
    xf = jnp.pad(xp.reshape(bpad, 1024), ((0, 0), (0, 4)))
    xs = jnp.stack([xf[:, kw:kw + _R1] for kw in range(5)], axis=2)
    xs = jnp.pad(xs, ((0, 0), (0, 0), (0, 3)))
    x5 = xs.reshape(bpad // 8, 8, _R1, 8).transpose(0, 2, 3, 1)
    x5 = x5.reshape(bpad // 8, _R1, 64)

    w1, w2 = _prep_weights(w1bd, w2bd)
    y2 = _fused_call(x5, w1, b1bd.reshape(1, 128), w2, b2bd.reshape(1, 128))
    xfc = y2.reshape(bpad // 4, 49, 4, 32).transpose(0, 2, 1, 3)
    xfc = xfc.reshape(bpad, 1568)
    out = _fc_call(xfc, fc_w.astype(jnp.bfloat16), fc_b)
    return out[:B, :10]


# speedup vs baseline: 1.0247x; 1.0247x over previous
import jax
import jax.numpy as jnp
from jax.experimental import pallas as pl
from jax.experimental.pallas import tpu as pltpu

_G = 16
_R1 = 1024
_WP2 = 24
_R2 = 432


def _round_up(x, m):
    return (x + m - 1) // m * m


def _fused_conv_kernel(x5_ref, w1_ref, b1_ref, w2_ref, b2_ref, o_ref,
                       patch1, acc1, x2i, acc2):
    x2i[...] = jnp.zeros(x2i.shape, x2i.dtype)

    for g in range(_G):
        for kh in range(5):
            patch1[:, pl.ds(kh * 64, 64)] = \
                x5_ref[g, pl.ds(kh * 32, 896), :]
        a1 = jnp.dot(patch1[...], w1_ref[...],
                     preferred_element_type=jnp.float32)
        acc1[...] = jnp.maximum(a1 + b1_ref[...], 0.0)

        for i in range(14):
            r0 = (2 * i) * 32
            r1 = r0 + 32
            p = jnp.maximum(
                jnp.maximum(acc1[pl.ds(r0, 14, 2), :],
                            acc1[pl.ds(r0 + 1, 14, 2), :]),
                jnp.maximum(acc1[pl.ds(r1, 14, 2), :],
                            acc1[pl.ds(r1 + 1, 14, 2), :]))
            pb = p.astype(jnp.bfloat16)
            for s in range(2):
                half = pb[:, s * 64:(s + 1) * 64]
                for kw in range(5):
                    x2i[s, pl.ds((i + 2) * _WP2 + 2 - kw, 14),
                        pl.ds(kw * 64, 64)] = half

        for s in range(2):
            for mh in range(2):
                acc = None
                for kh in range(5):
                    d = jnp.dot(x2i[s, pl.ds(kh * _WP2 + mh * 168, 168), :],
                                w2_ref[pl.ds(kh * 512, 512), :],
                                preferred_element_type=jnp.float32)
                    acc = d if acc is None else acc + d
                acc2[pl.ds(mh * 168, 168), :] = \
                    jnp.maximum(acc + b2_ref[...], 0.0)
            for i in range(7):
                r0 = (2 * i) * _WP2
                r1 = r0 + _WP2
                p = jnp.maximum(
                    jnp.maximum(acc2[pl.ds(r0, 7, 2), :],
                                acc2[pl.ds(r0 + 1, 7, 2), :]),
                    jnp.maximum(acc2[pl.ds(r1, 7, 2), :],
                                acc2[pl.ds(r1 + 1, 7, 2), :]))
                o_ref[g, s, pl.ds(i * 7, 7), :] = p.astype(o_ref.dtype)


def _fused_call(x5, w1, b1, w2, b2):
    nblk = x5.shape[0]
    return pl.pallas_call(
        _fused_conv_kernel,
        out_shape=jax.ShapeDtypeStruct((nblk, 2, 49, 128), jnp.bfloat16),
        grid=(nblk // _G,),
        in_specs=[
            pl.BlockSpec((_G, _R1, 64), lambda i: (i, 0, 0)),
            pl.BlockSpec((320, 128), lambda i: (0, 0)),
            pl.BlockSpec((1, 128), lambda i: (0, 0)),
            pl.BlockSpec((2560, 128), lambda i: (0, 0)),
            pl.BlockSpec((1, 128), lambda i: (0, 0)),
        ],
        out_specs=pl.BlockSpec((_G, 2, 49, 128), lambda i: (i, 0, 0, 0)),
        scratch_shapes=[
            pltpu.VMEM((896, 320), jnp.bfloat16),
            pltpu.VMEM((896, 128), jnp.float32),
            pltpu.VMEM((2, _R2, 512), jnp.bfloat16),
            pltpu.VMEM((336, 128), jnp.float32),
        ],
        compiler_params=pltpu.CompilerParams(
            dimension_semantics=("parallel",)),
    )(x5, w1, b1, w2, b2)


def _prep_weights(w1bd, w2bd):
    f32 = jnp.float32
    w1t = w1bd[:200].reshape(25, 8, 128)[:, 0, 0:16].reshape(5, 5, 16)
    t1 = jnp.einsum('hwc,dk->hwdkc', w1t, jnp.eye(8, dtype=f32))
    t1 = t1.reshape(5, 5, 8, 128).reshape(5, 40, 128)
    t1 = jnp.pad(t1, ((0, 0), (0, 24), (0, 0)))
    w1 = t1.reshape(320, 128).astype(jnp.bfloat16)
    w2t = w2bd.reshape(25, 4, 16, 128)[:, 0, :, 0:32].reshape(5, 5, 16, 32)
    t2 = jnp.einsum('hwxy,de->hwdxey', w2t, jnp.eye(4, dtype=f32))
    t2 = t2.reshape(5, 5, 64, 128)
    t2 = jnp.pad(t2, ((0, 0), (0, 3), (0, 0), (0, 0)))
    w2 = t2.reshape(5, 512, 128).reshape(2560, 128).astype(jnp.bfloat16)
    return w1, w2


def _fc_kernel(x_ref, w_ref, b_ref, o_ref):
    o_ref[...] = (jnp.dot(x_ref[...], w_ref[...],
                          preferred_element_type=jnp.float32)
                  + b_ref[...])


def _fc_call(x, w, bias, *, tile_m=512):
    M, K = x.shape
    _, N = w.shape
    tm = min(_round_up(M, 8), tile_m)
    Mp = _round_up(M, tm)
    if Mp != M:
        x = jnp.pad(x, ((0, Mp - M), (0, 0)))
    out = pl.pallas_call(
        _fc_kernel,
        out_shape=jax.ShapeDtypeStruct((Mp, N), jnp.float32),
        grid=(Mp // tm,),
        in_specs=[
            pl.BlockSpec((tm, K), lambda i: (i, 0)),
            pl.BlockSpec((K, N), lambda i: (0, 0)),
            pl.BlockSpec((1, N), lambda i: (0, 0)),
        ],
        out_specs=pl.BlockSpec((tm, N), lambda i: (i, 0)),
        compiler_params=pltpu.CompilerParams(
            dimension_semantics=("parallel",)),
    )(x, w, bias.reshape(1, N))
    return out[:M]


def kernel(w1bd, b1bd, w2bd, b2bd, fc_w, fc_b, x):
    B = x.shape[0]
    bpad = _round_up(B, 8 * _G)
    xb = x[:, 0, :, :]
    if bpad != B:
        xb = jnp.pad(xb, ((0, bpad - B), (0, 0), (0, 0)))
    xb = xb.astype(jnp.bfloat16)
    xp = jnp.pad(xb, ((0, 0), (2, 2), (2, 2)))
    xf = jnp.pad(xp.reshape(bpad, 1024), ((0, 0), (0, 4)))
    xs = jnp.stack([xf[:, kw:kw + _R1] for kw in range(5)], axis=2)
    xs = jnp.pad(xs, ((0, 0), (0, 0), (0, 3)))
    x5 = xs.reshape(bpad // 8, 8, _R1, 8).transpose(0, 2, 3, 1)
    x5 = x5.reshape(bpad // 8, _R1, 64)

    w1, w2 = _prep_weights(w1bd, w2bd)
    y2 = _fused_call(x5, w1, b1bd.reshape(1, 128), w2, b2bd.reshape(1, 128))
    xfc = y2.reshape(bpad // 4, 49, 4, 32).transpose(0, 2, 1, 3)
    xfc = xfc.reshape(bpad, 1568)
    out = _fc_call(xfc, fc_w.astype(jnp.bfloat16), fc_b)
    return out[:B, :10]
